# baseline (speedup 1.0000x reference)
import jax
import jax.numpy as jnp
from jax import lax
from jax.experimental import pallas as pl
from jax.experimental import pallas as _pl
from jax.experimental.pallas import tpu as pltpu

INTERP = 10
EPSILON = 0.01

QT = 256
NT = 2048


def _loss_kernel(qa_ref, s_ref, n_ref, out_ref, w_ref):
    j = pl.program_id(1)
    n_total = s_ref.shape[2]

    @pl.when(j == 0)
    def _():
        s0 = s_ref[0, 0:1, :]
        s1 = s_ref[0, 1:2, :]
        s2 = s_ref[0, 2:3, :]
        n0 = n_ref[0, 0:1, :]
        n1 = n_ref[0, 1:2, :]
        n2 = n_ref[0, 2:3, :]
        zero = jnp.zeros_like(s0)
        w_ref[0:1, :] = -2.0 * s0
        w_ref[1:2, :] = -2.0 * s1
        w_ref[2:3, :] = -2.0 * s2
        w_ref[3:4, :] = s0 * s0 + s1 * s1 + s2 * s2
        w_ref[4:5, :] = zero
        w_ref[5:6, :] = zero
        w_ref[6:7, :] = zero
        w_ref[7:8, :] = zero
        w_ref[8:9, :] = n0
        w_ref[9:10, :] = n1
        w_ref[10:11, :] = n2
        w_ref[11:12, :] = -(s0 * n0 + s1 * n1 + s2 * n2
                            + EPSILON * (n0 * n0 + n1 * n1 + n2 * n2))
        w_ref[12:13, :] = zero
        w_ref[13:14, :] = zero
        w_ref[14:15, :] = zero
        w_ref[15:16, :] = zero

    qa = qa_ref[0]

    def body(k, carry):
        run_min, run_dot = carry
        sl = pl.ds(k * NT, NT)
        wd = w_ref[0:8, sl]
        wn = w_ref[8:16, sl]
        d2m = jnp.dot(qa, wd, preferred_element_type=jnp.float32,
                      precision=lax.Precision.HIGHEST)
        dot = jnp.dot(qa, wn, preferred_element_type=jnp.float32,
                      precision=lax.Precision.HIGHEST)

        mn = jnp.min(d2m, axis=1, keepdims=True)
        dsel = jnp.min(jnp.where(d2m == mn, dot, jnp.inf),
                       axis=1, keepdims=True)

        upd = mn < run_min
        run_dot = jnp.where(upd, dsel, run_dot)
        run_min = jnp.where(upd, mn, run_min)
        return run_min, run_dot

    init = (jnp.full((QT, 1), jnp.inf, jnp.float32),
            jnp.zeros((QT, 1), jnp.float32))
    _, run_dot = lax.fori_loop(0, n_total // NT, body, init)

    loss = jnp.where(run_dot < 0.0, -run_dot, 0.0)
    part = jnp.sum(loss, axis=0, keepdims=True)

    i = pl.program_id(0)
    first = jnp.logical_and(i == 0, j == 0)
    last = jnp.logical_and(i == pl.num_programs(0) - 1,
                           j == pl.num_programs(1) - 1)

    @pl.when(first)
    def _():
        out_ref[...] = jnp.zeros((1, 1), jnp.float32)

    out_ref[...] += part

    @pl.when(last)
    def _():
        out_ref[...] = out_ref[...] * (1.0 / (pl.num_programs(0)
                                              * pl.num_programs(1) * QT))


def kernel(cage, shape, shape_normals):
    b, m, d = cage.shape
    n = shape.shape[1]
    q_count = m * INTERP

    cage_p = jnp.concatenate([cage[:, 1:, :], cage[:, :1, :]], axis=1)
    t = jnp.linspace(0.0, 1.0, INTERP).reshape(1, 1, INTERP, 1)
    q = (t * cage_p[:, :, None, :]
         + (1.0 - t) * cage[:, :, None, :]).reshape(b, q_count, d)
    qa = jnp.concatenate(
        [q, jnp.ones((b, q_count, 1), jnp.float32),
         jnp.zeros((b, q_count, 4), jnp.float32)], axis=-1)

    shape_t = shape.transpose(0, 2, 1)
    normals_t = shape_normals.transpose(0, 2, 1)

    out = pl.pallas_call(
        _loss_kernel,
        grid=(b, q_count // QT),
        in_specs=[
            pl.BlockSpec((1, QT, 8), lambda i, j: (i, j, 0)),
            pl.BlockSpec((1, d, n), lambda i, j: (i, 0, 0)),
            pl.BlockSpec((1, d, n), lambda i, j: (i, 0, 0)),
        ],
        out_specs=pl.BlockSpec((1, 1), lambda i, j: (0, 0)),
        out_shape=jax.ShapeDtypeStruct((1, 1), jnp.float32),
        scratch_shapes=[pltpu.VMEM((16, n), jnp.float32)],
    )(qa, shape_t, normals_t)
    return out[0, 0]


# speedup vs baseline: 2.3418x; 2.3418x over previous
import jax
import jax.numpy as jnp
from jax import lax
from jax.experimental import pallas as pl
from jax.experimental.pallas import tpu as pltpu

INTERP = 10
EPSILON = 0.01

QT = 256
NT = 2048


def _loss_kernel(q_ref, s_ref, n_ref, out_ref, w_ref):
    j = pl.program_id(1)
    n_total = s_ref.shape[2]

    @pl.when(j == 0)
    def _():
        s0 = s_ref[0, 0:1, :]
        s1 = s_ref[0, 1:2, :]
        s2 = s_ref[0, 2:3, :]
        n0 = n_ref[0, 0:1, :]
        n1 = n_ref[0, 1:2, :]
        n2 = n_ref[0, 2:3, :]
        w_ref[0:1, :] = -2.0 * s0
        w_ref[1:2, :] = -2.0 * s1
        w_ref[2:3, :] = -2.0 * s2
        w_ref[3:4, :] = s0 * s0 + s1 * s1 + s2 * s2
        w_ref[4:5, :] = n0
        w_ref[5:6, :] = n1
        w_ref[6:7, :] = n2
        w_ref[7:8, :] = -(s0 * n0 + s1 * n1 + s2 * n2
                          + EPSILON * (n0 * n0 + n1 * n1 + n2 * n2))

    qb = q_ref[0]
    q0 = qb[:, 0:1]
    q1 = qb[:, 1:2]
    q2 = qb[:, 2:3]

    def body(k, carry):
        run_min, run_dot = carry
        sl = pl.ds(k * NT, NT)
        s0m = w_ref[0:1, sl]
        s1m = w_ref[1:2, sl]
        s2m = w_ref[2:3, sl]
        ss = w_ref[3:4, sl]
        n0 = w_ref[4:5, sl]
        n1 = w_ref[5:6, sl]
        n2 = w_ref[6:7, sl]
        nc = w_ref[7:8, sl]

        d2m = q2 * s2m + (q1 * s1m + (q0 * s0m + ss))
        dot = q2 * n2 + (q1 * n1 + (q0 * n0 + nc))

        mn = jnp.min(d2m, axis=1, keepdims=True)
        dsel = jnp.min(jnp.where(d2m == mn, dot, jnp.inf),
                       axis=1, keepdims=True)

        upd = mn < run_min
        run_dot = jnp.where(upd, dsel, run_dot)
        run_min = jnp.where(upd, mn, run_min)
        return run_min, run_dot

    init = (jnp.full((QT, 1), jnp.inf, jnp.float32),
            jnp.zeros((QT, 1), jnp.float32))
    _, run_dot = lax.fori_loop(0, n_total // NT, body, init)

    loss = jnp.where(run_dot < 0.0, -run_dot, 0.0)
    part = jnp.sum(loss, axis=0, keepdims=True)

    i = pl.program_id(0)
    first = jnp.logical_and(i == 0, j == 0)
    last = jnp.logical_and(i == pl.num_programs(0) - 1,
                           j == pl.num_programs(1) - 1)

    @pl.when(first)
    def _():
        out_ref[...] = jnp.zeros((1, 1), jnp.float32)

    out_ref[...] += part

    @pl.when(last)
    def _():
        out_ref[...] = out_ref[...] * (1.0 / (pl.num_programs(0)
                                              * pl.num_programs(1) * QT))


def kernel(cage, shape, shape_normals):
    b, m, d = cage.shape
    n = shape.shape[1]
    q_count = m * INTERP

    cage_p = jnp.concatenate([cage[:, 1:, :], cage[:, :1, :]], axis=1)
    t = jnp.linspace(0.0, 1.0, INTERP).reshape(1, 1, INTERP, 1)
    q = (t * cage_p[:, :, None, :]
         + (1.0 - t) * cage[:, :, None, :]).reshape(b, q_count, d)

    shape_t = shape.transpose(0, 2, 1)
    normals_t = shape_normals.transpose(0, 2, 1)

    out = pl.pallas_call(
        _loss_kernel,
        grid=(b, q_count // QT),
        in_specs=[
            pl.BlockSpec((1, QT, d), lambda i, j: (i, j, 0)),
            pl.BlockSpec((1, d, n), lambda i, j: (i, 0, 0)),
            pl.BlockSpec((1, d, n), lambda i, j: (i, 0, 0)),
        ],
        out_specs=pl.BlockSpec((1, 1), lambda i, j: (0, 0)),
        out_shape=jax.ShapeDtypeStruct((1, 1), jnp.float32),
        scratch_shapes=[pltpu.VMEM((8, n), jnp.float32)],
    )(q, shape_t, normals_t)
    return out[0, 0]


# speedup vs baseline: 2.3758x; 1.0145x over previous
import functools

import jax
import jax.numpy as jnp
from jax import lax
from jax.experimental import pallas as pl
from jax.experimental.pallas import tpu as pltpu
from jax.experimental.pallas import tpu_sc as plsc

INTERP = 10
EPSILON = 0.01

QT = 256
NT = 2048

NW = 32
LANES = 16


def _argmin_kernel(q_ref, s_ref, out_ref, w_ref):
    i = pl.program_id(0)
    j = pl.program_id(1)
    n_total = s_ref.shape[2]

    @pl.when(j == 0)
    def _():
        s0 = s_ref[0, 0:1, :]
        s1 = s_ref[0, 1:2, :]
        s2 = s_ref[0, 2:3, :]
        w_ref[0:1, :] = -2.0 * s0
        w_ref[1:2, :] = -2.0 * s1
        w_ref[2:3, :] = -2.0 * s2
        w_ref[3:4, :] = s0 * s0 + s1 * s1 + s2 * s2

    qb = q_ref[0]
    q0 = qb[:, 0:1]
    q1 = qb[:, 1:2]
    q2 = qb[:, 2:3]

    col = lax.broadcasted_iota(jnp.int32, (QT, NT), 1)

    def body(k, carry):
        run_min, run_loc, run_chunk = carry
        sl = pl.ds(k * NT, NT)
        s0m = w_ref[0:1, sl]
        s1m = w_ref[1:2, sl]
        s2m = w_ref[2:3, sl]
        ss = w_ref[3:4, sl]

        d2m = q2 * s2m + (q1 * s1m + (q0 * s0m + ss))
        mn = jnp.min(d2m, axis=1, keepdims=True)
        loc = jnp.min(jnp.where(d2m == mn, col, NT),
                      axis=1, keepdims=True)

        upd = mn < run_min
        run_loc = jnp.where(upd, loc, run_loc)
        run_chunk = jnp.where(upd, k, run_chunk)
        run_min = jnp.where(upd, mn, run_min)
        return run_min, run_loc, run_chunk

    init = (jnp.full((QT, 1), jnp.inf, jnp.float32),
            jnp.zeros((QT, 1), jnp.int32),
            jnp.zeros((QT, 1), jnp.int32))
    _, run_loc, run_chunk = lax.fori_loop(0, n_total // NT, body, init)

    idx = run_chunk * NT + run_loc + i * n_total
    out_ref[...] = idx[None]


CHUNK = 64


def _make_sc_loss(total_rows, total_q):
    pw = total_q // NW
    groups = pw // LANES
    chunks = pw // CHUNK
    per_chunk = CHUNK // LANES
    mesh = plsc.VectorSubcoreMesh(core_axis_name="c", subcore_axis_name="s")

    @functools.partial(
        pl.kernel, mesh=mesh,
        out_type=jax.ShapeDtypeStruct((NW, LANES), jnp.float32),
        scratch_types=[
            pltpu.VMEM((chunks, CHUNK), jnp.int32),
            pltpu.VMEM((pw,), jnp.float32),
            pltpu.VMEM((pw,), jnp.float32),
            pltpu.VMEM((pw,), jnp.float32),
            pltpu.VMEM((chunks, CHUNK), jnp.float32),
            pltpu.VMEM((chunks, CHUNK), jnp.float32),
            pltpu.VMEM((chunks, CHUNK), jnp.float32),
            pltpu.VMEM((chunks, CHUNK), jnp.float32),
            pltpu.VMEM((chunks, CHUNK), jnp.float32),
            pltpu.VMEM((chunks, CHUNK), jnp.float32),
            pltpu.VMEM((LANES,), jnp.float32),
            pltpu.SemaphoreType.DMA,
        ],
    )
    def sc_loss(idx_hbm, q0_hbm, q1_hbm, q2_hbm,
                s0_hbm, s1_hbm, s2_hbm, n0_hbm, n1_hbm, n2_hbm,
                out_hbm,
                idx_v, q0_v, q1_v, q2_v,
                s0_g, s1_g, s2_g, n0_g, n1_g, n2_g,
                acc_v, sem):
        wid = lax.axis_index("s") * 2 + lax.axis_index("c")
        base = wid * pw

        for c in range(chunks):
            pltpu.sync_copy(idx_hbm.at[pl.ds(base + c * CHUNK, CHUNK)],
                            idx_v.at[c])
        pltpu.sync_copy(q0_hbm.at[pl.ds(base, pw)], q0_v)
        pltpu.sync_copy(q1_hbm.at[pl.ds(base, pw)], q1_v)
        pltpu.sync_copy(q2_hbm.at[pl.ds(base, pw)], q2_v)

        copies = []
        for c in range(chunks):
            for hbm, dst in ((s0_hbm, s0_g), (s1_hbm, s1_g), (s2_hbm, s2_g),
                             (n0_hbm, n0_g), (n1_hbm, n1_g), (n2_hbm, n2_g)):
                copies.append(
                    pltpu.async_copy(hbm.at[idx_v.at[c]], dst.at[c], sem))
        for cp in copies:
            cp.wait()

        acc = jnp.zeros((LANES,), jnp.float32)
        for g in range(groups):
            r = g // per_chunk
            sl2 = pl.ds((g % per_chunk) * LANES, LANES)
            sl = pl.ds(g * LANES, LANES)
            p0 = s0_g[r, sl2]
            p1 = s1_g[r, sl2]
            p2 = s2_g[r, sl2]
            m0 = n0_g[r, sl2]
            m1 = n1_g[r, sl2]
            m2 = n2_g[r, sl2]
            dot = (((q0_v[sl] - p0) - EPSILON * m0) * m0
                   + ((q1_v[sl] - p1) - EPSILON * m1) * m1
                   + ((q2_v[sl] - p2) - EPSILON * m2) * m2)
            acc = acc + jnp.maximum(-dot, 0.0)

        acc_v[...] = acc
        pltpu.sync_copy(acc_v, out_hbm.at[wid])

    return sc_loss


def kernel(cage, shape, shape_normals):
    b, m, d = cage.shape
    n = shape.shape[1]
    q_count = m * INTERP

    cage_p = jnp.concatenate([cage[:, 1:, :], cage[:, :1, :]], axis=1)
    t = jnp.linspace(0.0, 1.0, INTERP).reshape(1, 1, INTERP, 1)
    q = (t * cage_p[:, :, None, :]
         + (1.0 - t) * cage[:, :, None, :]).reshape(b, q_count, d)

    shape_t = shape.transpose(0, 2, 1)

    idx = pl.pallas_call(
        _argmin_kernel,
        grid=(b, q_count // QT),
        in_specs=[
            pl.BlockSpec((1, QT, d), lambda i, j: (i, j, 0)),
            pl.BlockSpec((1, d, n), lambda i, j: (i, 0, 0)),
        ],
        out_specs=pl.BlockSpec((1, QT, 1), lambda i, j: (i, j, 0)),
        out_shape=jax.ShapeDtypeStruct((b, q_count, 1), jnp.int32),
        scratch_shapes=[pltpu.VMEM((4, n), jnp.float32)],
    )(q, shape_t)

    idx_flat = idx.reshape(b * q_count)
    q0 = q[:, :, 0].reshape(-1)
    q1 = q[:, :, 1].reshape(-1)
    q2 = q[:, :, 2].reshape(-1)
    s0 = shape[:, :, 0].reshape(-1)
    s1 = shape[:, :, 1].reshape(-1)
    s2 = shape[:, :, 2].reshape(-1)
    n0 = shape_normals[:, :, 0].reshape(-1)
    n1 = shape_normals[:, :, 1].reshape(-1)
    n2 = shape_normals[:, :, 2].reshape(-1)

    sc_loss = _make_sc_loss(b * n, b * q_count)
    partials = sc_loss(idx_flat, q0, q1, q2, s0, s1, s2, n0, n1, n2)

    return jnp.sum(partials) / (b * q_count)


# speedup vs baseline: 3.3836x; 1.4242x over previous
import functools

import jax
import jax.numpy as jnp
from jax import lax
from jax.experimental import pallas as pl
from jax.experimental.pallas import tpu as pltpu
from jax.experimental.pallas import tpu_sc as plsc

INTERP = 10
EPSILON = 0.01

QT = 256
NT = 2048

NW = 32
LANES = 16


def _argmin_kernel(q_ref, w_ref, out_ref):
    i = pl.program_id(0)
    n_total = w_ref.shape[2]

    qa = q_ref[0]

    colf = lax.broadcasted_iota(jnp.int32, (QT, NT), 1).astype(jnp.float32)

    def body(k, carry):
        run_min, run_loc, run_chunk = carry
        sl = pl.ds(k * NT, NT)
        d2m = jnp.dot(qa, w_ref[0, :, sl],
                      preferred_element_type=jnp.float32)
        mn = jnp.min(d2m, axis=1, keepdims=True)
        loc = jnp.min(jnp.where(d2m == mn, colf, float(NT)),
                      axis=1, keepdims=True)

        upd = mn < run_min
        run_loc = jnp.where(upd, loc, run_loc)
        run_chunk = jnp.where(upd, k.astype(jnp.float32), run_chunk)
        run_min = jnp.where(upd, mn, run_min)
        return run_min, run_loc, run_chunk

    init = (jnp.full((QT, 1), jnp.inf, jnp.float32),
            jnp.zeros((QT, 1), jnp.float32),
            jnp.zeros((QT, 1), jnp.float32))
    _, run_loc, run_chunk = lax.fori_loop(0, n_total // NT, body, init)

    idx = (run_chunk * float(NT) + run_loc).astype(jnp.int32) + i * n_total
    out_ref[...] = idx[None]


CHUNK = 64


def _make_sc_loss(total_rows, total_q):
    pw = total_q // NW
    groups = pw // LANES
    chunks = pw // CHUNK
    per_chunk = CHUNK // LANES
    mesh = plsc.VectorSubcoreMesh(core_axis_name="c", subcore_axis_name="s")

    @functools.partial(
        pl.kernel, mesh=mesh,
        out_type=jax.ShapeDtypeStruct((NW, LANES), jnp.float32),
        scratch_types=[
            pltpu.VMEM((chunks, CHUNK), jnp.int32),
            pltpu.VMEM((pw,), jnp.float32),
            pltpu.VMEM((pw,), jnp.float32),
            pltpu.VMEM((pw,), jnp.float32),
            pltpu.VMEM((chunks, CHUNK), jnp.float32),
            pltpu.VMEM((chunks, CHUNK), jnp.float32),
            pltpu.VMEM((chunks, CHUNK), jnp.float32),
            pltpu.VMEM((chunks, CHUNK), jnp.float32),
            pltpu.VMEM((chunks, CHUNK), jnp.float32),
            pltpu.VMEM((chunks, CHUNK), jnp.float32),
            pltpu.VMEM((LANES,), jnp.float32),
            pltpu.SemaphoreType.DMA,
        ],
    )
    def sc_loss(idx_hbm, q0_hbm, q1_hbm, q2_hbm,
                s0_hbm, s1_hbm, s2_hbm, n0_hbm, n1_hbm, n2_hbm,
                out_hbm,
                idx_v, q0_v, q1_v, q2_v,
                s0_g, s1_g, s2_g, n0_g, n1_g, n2_g,
                acc_v, sem):
        wid = lax.axis_index("s") * 2 + lax.axis_index("c")
        base = wid * pw

        for c in range(chunks):
            pltpu.sync_copy(idx_hbm.at[pl.ds(base + c * CHUNK, CHUNK)],
                            idx_v.at[c])
        pltpu.sync_copy(q0_hbm.at[pl.ds(base, pw)], q0_v)
        pltpu.sync_copy(q1_hbm.at[pl.ds(base, pw)], q1_v)
        pltpu.sync_copy(q2_hbm.at[pl.ds(base, pw)], q2_v)

        copies = []
        for c in range(chunks):
            for hbm, dst in ((s0_hbm, s0_g), (s1_hbm, s1_g), (s2_hbm, s2_g),
                             (n0_hbm, n0_g), (n1_hbm, n1_g), (n2_hbm, n2_g)):
                copies.append(
                    pltpu.async_copy(hbm.at[idx_v.at[c]], dst.at[c], sem))
        for cp in copies:
            cp.wait()

        acc = jnp.zeros((LANES,), jnp.float32)
        for g in range(groups):
            r = g // per_chunk
            sl2 = pl.ds((g % per_chunk) * LANES, LANES)
            sl = pl.ds(g * LANES, LANES)
            p0 = s0_g[r, sl2]
            p1 = s1_g[r, sl2]
            p2 = s2_g[r, sl2]
            m0 = n0_g[r, sl2]
            m1 = n1_g[r, sl2]
            m2 = n2_g[r, sl2]
            dot = (((q0_v[sl] - p0) - EPSILON * m0) * m0
                   + ((q1_v[sl] - p1) - EPSILON * m1) * m1
                   + ((q2_v[sl] - p2) - EPSILON * m2) * m2)
            acc = acc + jnp.maximum(-dot, 0.0)

        acc_v[...] = acc
        pltpu.sync_copy(acc_v, out_hbm.at[wid])

    return sc_loss


def kernel(cage, shape, shape_normals):
    b, m, d = cage.shape
    n = shape.shape[1]
    q_count = m * INTERP

    cage_p = jnp.concatenate([cage[:, 1:, :], cage[:, :1, :]], axis=1)
    t = jnp.linspace(0.0, 1.0, INTERP).reshape(1, 1, INTERP, 1)
    q = (t * cage_p[:, :, None, :]
         + (1.0 - t) * cage[:, :, None, :]).reshape(b, q_count, d)

    def _split(x):
        bits = lax.bitcast_convert_type(x, jnp.uint32)
        hi_f = lax.bitcast_convert_type(
            bits & jnp.uint32(0xFFFF0000), jnp.float32)
        return hi_f.astype(jnp.bfloat16), (x - hi_f).astype(jnp.bfloat16)

    shape_t = shape.transpose(0, 2, 1)
    ss = jnp.sum(shape_t * shape_t, axis=1, keepdims=True)
    wf = jnp.concatenate([-2.0 * shape_t, ss,
                          jnp.zeros((b, 4, n), jnp.float32)], axis=1)
    w_hi, w_lo = _split(wf)
    w_c = jnp.concatenate(
        [w_hi, w_lo, w_hi,
         jnp.zeros((b, 8, n), jnp.bfloat16)], axis=1)

    qf = jnp.concatenate(
        [q, jnp.ones((b, q_count, 1), jnp.float32),
         jnp.zeros((b, q_count, 4), jnp.float32)], axis=-1)
    q_hi, q_lo = _split(qf)
    qa = jnp.concatenate(
        [q_hi, q_hi, q_lo,
         jnp.zeros((b, q_count, 8), jnp.bfloat16)], axis=-1)

    idx = pl.pallas_call(
        _argmin_kernel,
        grid=(b, q_count // QT),
        in_specs=[
            pl.BlockSpec((1, QT, 32), lambda i, j: (i, j, 0)),
            pl.BlockSpec((1, 32, n), lambda i, j: (i, 0, 0)),
        ],
        out_specs=pl.BlockSpec((1, QT, 1), lambda i, j: (i, j, 0)),
        out_shape=jax.ShapeDtypeStruct((b, q_count, 1), jnp.int32),
    )(qa, w_c)

    idx_flat = idx.reshape(b * q_count)
    q0 = q[:, :, 0].reshape(-1)
    q1 = q[:, :, 1].reshape(-1)
    q2 = q[:, :, 2].reshape(-1)
    s0 = shape[:, :, 0].reshape(-1)
    s1 = shape[:, :, 1].reshape(-1)
    s2 = shape[:, :, 2].reshape(-1)
    n0 = shape_normals[:, :, 0].reshape(-1)
    n1 = shape_normals[:, :, 1].reshape(-1)
    n2 = shape_normals[:, :, 2].reshape(-1)

    sc_loss = _make_sc_loss(b * n, b * q_count)
    partials = sc_loss(idx_flat, q0, q1, q2, s0, s1, s2, n0, n1, n2)

    return jnp.sum(partials) / (b * q_count)


# speedup vs baseline: 3.8277x; 1.1312x over previous
import functools

import jax
import jax.numpy as jnp
from jax import lax
from jax.experimental import pallas as pl
from jax.experimental.pallas import tpu as pltpu
from jax.experimental.pallas import tpu_sc as plsc

INTERP = 10
EPSILON = 0.01

QT = 512
NT = 2048

NW = 32
LANES = 16


def _argmin_kernel(q_ref, w_ref, out_ref):
    i = pl.program_id(0)
    n_total = w_ref.shape[2]

    qa = q_ref[0]

    colf = lax.broadcasted_iota(jnp.int32, (QT, NT), 1).astype(jnp.float32)

    def body(k, carry):
        run_min, run_loc, run_chunk = carry
        sl = pl.ds(k * NT, NT)
        d2m = jnp.dot(qa, w_ref[0, :, sl],
                      preferred_element_type=jnp.float32)
        mn = jnp.min(d2m, axis=1, keepdims=True)
        loc = jnp.min(jnp.where(d2m == mn, colf, float(NT)),
                      axis=1, keepdims=True)

        upd = mn < run_min
        run_loc = jnp.where(upd, loc, run_loc)
        run_chunk = jnp.where(upd, k.astype(jnp.float32), run_chunk)
        run_min = jnp.where(upd, mn, run_min)
        return run_min, run_loc, run_chunk

    init = (jnp.full((QT, 1), jnp.inf, jnp.float32),
            jnp.zeros((QT, 1), jnp.float32),
            jnp.zeros((QT, 1), jnp.float32))
    _, run_loc, run_chunk = lax.fori_loop(0, n_total // NT, body, init)

    idx = (run_chunk * float(NT) + run_loc).astype(jnp.int32) + i * n_total
    out_ref[...] = idx[None]


CHUNK = 64


def _make_sc_loss(total_rows, total_q):
    pw = total_q // NW
    groups = pw // LANES
    chunks = pw // CHUNK
    per_chunk = CHUNK // LANES
    mesh = plsc.VectorSubcoreMesh(core_axis_name="c", subcore_axis_name="s")

    @functools.partial(
        pl.kernel, mesh=mesh,
        out_type=jax.ShapeDtypeStruct((NW, LANES), jnp.float32),
        scratch_types=[
            pltpu.VMEM((chunks, CHUNK), jnp.int32),
            pltpu.VMEM((pw,), jnp.float32),
            pltpu.VMEM((pw,), jnp.float32),
            pltpu.VMEM((pw,), jnp.float32),
            pltpu.VMEM((chunks, CHUNK), jnp.float32),
            pltpu.VMEM((chunks, CHUNK), jnp.float32),
            pltpu.VMEM((chunks, CHUNK), jnp.float32),
            pltpu.VMEM((chunks, CHUNK), jnp.float32),
            pltpu.VMEM((chunks, CHUNK), jnp.float32),
            pltpu.VMEM((chunks, CHUNK), jnp.float32),
            pltpu.VMEM((LANES,), jnp.float32),
            pltpu.SemaphoreType.DMA,
        ],
    )
    def sc_loss(idx_hbm, q0_hbm, q1_hbm, q2_hbm,
                s0_hbm, s1_hbm, s2_hbm, n0_hbm, n1_hbm, n2_hbm,
                out_hbm,
                idx_v, q0_v, q1_v, q2_v,
                s0_g, s1_g, s2_g, n0_g, n1_g, n2_g,
                acc_v, sem):
        wid = lax.axis_index("s") * 2 + lax.axis_index("c")
        base = wid * pw

        for c in range(chunks):
            pltpu.sync_copy(idx_hbm.at[pl.ds(base + c * CHUNK, CHUNK)],
                            idx_v.at[c])
        pltpu.sync_copy(q0_hbm.at[pl.ds(base, pw)], q0_v)
        pltpu.sync_copy(q1_hbm.at[pl.ds(base, pw)], q1_v)
        pltpu.sync_copy(q2_hbm.at[pl.ds(base, pw)], q2_v)

        copies = []
        for c in range(chunks):
            for hbm, dst in ((s0_hbm, s0_g), (s1_hbm, s1_g), (s2_hbm, s2_g),
                             (n0_hbm, n0_g), (n1_hbm, n1_g), (n2_hbm, n2_g)):
                copies.append(
                    pltpu.async_copy(hbm.at[idx_v.at[c]], dst.at[c], sem))
        for cp in copies:
            cp.wait()

        acc = jnp.zeros((LANES,), jnp.float32)
        for g in range(groups):
            r = g // per_chunk
            sl2 = pl.ds((g % per_chunk) * LANES, LANES)
            sl = pl.ds(g * LANES, LANES)
            p0 = s0_g[r, sl2]
            p1 = s1_g[r, sl2]
            p2 = s2_g[r, sl2]
            m0 = n0_g[r, sl2]
            m1 = n1_g[r, sl2]
            m2 = n2_g[r, sl2]
            dot = (((q0_v[sl] - p0) - EPSILON * m0) * m0
                   + ((q1_v[sl] - p1) - EPSILON * m1) * m1
                   + ((q2_v[sl] - p2) - EPSILON * m2) * m2)
            acc = acc + jnp.maximum(-dot, 0.0)

        acc_v[...] = acc
        pltpu.sync_copy(acc_v, out_hbm.at[wid])

    return sc_loss


def kernel(cage, shape, shape_normals):
    b, m, d = cage.shape
    n = shape.shape[1]
    q_count = m * INTERP

    cage_p = jnp.concatenate([cage[:, 1:, :], cage[:, :1, :]], axis=1)
    t = jnp.linspace(0.0, 1.0, INTERP).reshape(1, 1, INTERP, 1)
    q = (t * cage_p[:, :, None, :]
         + (1.0 - t) * cage[:, :, None, :]).reshape(b, q_count, d)

    def _split(x):
        bits = lax.bitcast_convert_type(x, jnp.uint32)
        hi_f = lax.bitcast_convert_type(
            bits & jnp.uint32(0xFFFF0000), jnp.float32)
        return hi_f.astype(jnp.bfloat16), (x - hi_f).astype(jnp.bfloat16)

    shape_t = shape.transpose(0, 2, 1)
    ss = jnp.sum(shape_t * shape_t, axis=1, keepdims=True)
    wf = jnp.concatenate([-2.0 * shape_t, ss,
                          jnp.zeros((b, 4, n), jnp.float32)], axis=1)
    w_hi, w_lo = _split(wf)
    w_c = jnp.concatenate(
        [w_hi, w_lo, w_hi,
         jnp.zeros((b, 8, n), jnp.bfloat16)], axis=1)

    qf = jnp.concatenate(
        [q, jnp.ones((b, q_count, 1), jnp.float32),
         jnp.zeros((b, q_count, 4), jnp.float32)], axis=-1)
    q_hi, q_lo = _split(qf)
    qa = jnp.concatenate(
        [q_hi, q_hi, q_lo,
         jnp.zeros((b, q_count, 8), jnp.bfloat16)], axis=-1)

    idx = pl.pallas_call(
        _argmin_kernel,
        grid=(b, q_count // QT),
        in_specs=[
            pl.BlockSpec((1, QT, 32), lambda i, j: (i, j, 0)),
            pl.BlockSpec((1, 32, n), lambda i, j: (i, 0, 0)),
        ],
        out_specs=pl.BlockSpec((1, QT, 1), lambda i, j: (i, j, 0)),
        out_shape=jax.ShapeDtypeStruct((b, q_count, 1), jnp.int32),
    )(qa, w_c)

    idx_flat = idx.reshape(b * q_count)
    q0 = q[:, :, 0].reshape(-1)
    q1 = q[:, :, 1].reshape(-1)
    q2 = q[:, :, 2].reshape(-1)
    s0 = shape[:, :, 0].reshape(-1)
    s1 = shape[:, :, 1].reshape(-1)
    s2 = shape[:, :, 2].reshape(-1)
    n0 = shape_normals[:, :, 0].reshape(-1)
    n1 = shape_normals[:, :, 1].reshape(-1)
    n2 = shape_normals[:, :, 2].reshape(-1)

    sc_loss = _make_sc_loss(b * n, b * q_count)
    partials = sc_loss(idx_flat, q0, q1, q2, s0, s1, s2, n0, n1, n2)

    return jnp.sum(partials) / (b * q_count)


# speedup vs baseline: 4.1020x; 1.0717x over previous
import functools

import jax
import jax.numpy as jnp
from jax import lax
from jax.experimental import pallas as pl
from jax.experimental.pallas import tpu as pltpu
from jax.experimental.pallas import tpu_sc as plsc

INTERP = 10
EPSILON = 0.01

QT = 512
NT = 4096

NW = 32
LANES = 16


def _argmin_kernel(q_ref, w_ref, out_ref):
    i = pl.program_id(0)
    n_total = w_ref.shape[2]

    qa = q_ref[0]

    colf = lax.broadcasted_iota(jnp.int32, (QT, NT), 1).astype(jnp.float32)

    def body(k, carry):
        run_min, run_loc, run_chunk = carry
        sl = pl.ds(k * NT, NT)
        d2m = jnp.dot(qa, w_ref[0, :, sl],
                      preferred_element_type=jnp.float32)
        mn = jnp.min(d2m, axis=1, keepdims=True)
        loc = jnp.min(jnp.where(d2m == mn, colf, float(NT)),
                      axis=1, keepdims=True)

        upd = mn < run_min
        run_loc = jnp.where(upd, loc, run_loc)
        run_chunk = jnp.where(upd, k.astype(jnp.float32), run_chunk)
        run_min = jnp.where(upd, mn, run_min)
        return run_min, run_loc, run_chunk

    init = (jnp.full((QT, 1), jnp.inf, jnp.float32),
            jnp.zeros((QT, 1), jnp.float32),
            jnp.zeros((QT, 1), jnp.float32))
    _, run_loc, run_chunk = lax.fori_loop(0, n_total // NT, body, init)

    idx = (run_chunk * float(NT) + run_loc).astype(jnp.int32) + i * n_total
    out_ref[...] = idx[None]


CHUNK = 64


def _make_sc_loss(total_rows, total_q):
    pw = total_q // NW
    groups = pw // LANES
    chunks = pw // CHUNK
    per_chunk = CHUNK // LANES
    mesh = plsc.VectorSubcoreMesh(core_axis_name="c", subcore_axis_name="s")

    @functools.partial(
        pl.kernel, mesh=mesh,
        out_type=jax.ShapeDtypeStruct((NW, LANES), jnp.float32),
        scratch_types=[
            pltpu.VMEM((chunks, CHUNK), jnp.int32),
            pltpu.VMEM((pw,), jnp.float32),
            pltpu.VMEM((pw,), jnp.float32),
            pltpu.VMEM((pw,), jnp.float32),
            pltpu.VMEM((chunks, CHUNK), jnp.float32),
            pltpu.VMEM((chunks, CHUNK), jnp.float32),
            pltpu.VMEM((chunks, CHUNK), jnp.float32),
            pltpu.VMEM((chunks, CHUNK), jnp.float32),
            pltpu.VMEM((chunks, CHUNK), jnp.float32),
            pltpu.VMEM((chunks, CHUNK), jnp.float32),
            pltpu.VMEM((LANES,), jnp.float32),
            pltpu.SemaphoreType.DMA,
        ],
    )
    def sc_loss(idx_hbm, q0_hbm, q1_hbm, q2_hbm,
                s0_hbm, s1_hbm, s2_hbm, n0_hbm, n1_hbm, n2_hbm,
                out_hbm,
                idx_v, q0_v, q1_v, q2_v,
                s0_g, s1_g, s2_g, n0_g, n1_g, n2_g,
                acc_v, sem):
        wid = lax.axis_index("s") * 2 + lax.axis_index("c")
        base = wid * pw

        for c in range(chunks):
            pltpu.sync_copy(idx_hbm.at[pl.ds(base + c * CHUNK, CHUNK)],
                            idx_v.at[c])
        pltpu.sync_copy(q0_hbm.at[pl.ds(base, pw)], q0_v)
        pltpu.sync_copy(q1_hbm.at[pl.ds(base, pw)], q1_v)
        pltpu.sync_copy(q2_hbm.at[pl.ds(base, pw)], q2_v)

        copies = []
        for c in range(chunks):
            for hbm, dst in ((s0_hbm, s0_g), (s1_hbm, s1_g), (s2_hbm, s2_g),
                             (n0_hbm, n0_g), (n1_hbm, n1_g), (n2_hbm, n2_g)):
                copies.append(
                    pltpu.async_copy(hbm.at[idx_v.at[c]], dst.at[c], sem))
        for cp in copies:
            cp.wait()

        acc = jnp.zeros((LANES,), jnp.float32)
        for g in range(groups):
            r = g // per_chunk
            sl2 = pl.ds((g % per_chunk) * LANES, LANES)
            sl = pl.ds(g * LANES, LANES)
            p0 = s0_g[r, sl2]
            p1 = s1_g[r, sl2]
            p2 = s2_g[r, sl2]
            m0 = n0_g[r, sl2]
            m1 = n1_g[r, sl2]
            m2 = n2_g[r, sl2]
            dot = (((q0_v[sl] - p0) - EPSILON * m0) * m0
                   + ((q1_v[sl] - p1) - EPSILON * m1) * m1
                   + ((q2_v[sl] - p2) - EPSILON * m2) * m2)
            acc = acc + jnp.maximum(-dot, 0.0)

        acc_v[...] = acc
        pltpu.sync_copy(acc_v, out_hbm.at[wid])

    return sc_loss


def kernel(cage, shape, shape_normals):
    b, m, d = cage.shape
    n = shape.shape[1]
    q_count = m * INTERP

    cage_p = jnp.concatenate([cage[:, 1:, :], cage[:, :1, :]], axis=1)
    t = jnp.linspace(0.0, 1.0, INTERP).reshape(1, 1, INTERP, 1)
    q = (t * cage_p[:, :, None, :]
         + (1.0 - t) * cage[:, :, None, :]).reshape(b, q_count, d)

    def _split(x):
        bits = lax.bitcast_convert_type(x, jnp.uint32)
        hi_f = lax.bitcast_convert_type(
            bits & jnp.uint32(0xFFFF0000), jnp.float32)
        return hi_f.astype(jnp.bfloat16), (x - hi_f).astype(jnp.bfloat16)

    shape_t = shape.transpose(0, 2, 1)
    ss = jnp.sum(shape_t * shape_t, axis=1, keepdims=True)
    wf = jnp.concatenate([-2.0 * shape_t, ss,
                          jnp.zeros((b, 4, n), jnp.float32)], axis=1)
    w_hi, w_lo = _split(wf)
    w_c = jnp.concatenate(
        [w_hi, w_lo, w_hi,
         jnp.zeros((b, 8, n), jnp.bfloat16)], axis=1)

    qf = jnp.concatenate(
        [q, jnp.ones((b, q_count, 1), jnp.float32),
         jnp.zeros((b, q_count, 4), jnp.float32)], axis=-1)
    q_hi, q_lo = _split(qf)
    qa = jnp.concatenate(
        [q_hi, q_hi, q_lo,
         jnp.zeros((b, q_count, 8), jnp.bfloat16)], axis=-1)

    idx = pl.pallas_call(
        _argmin_kernel,
        grid=(b, q_count // QT),
        in_specs=[
            pl.BlockSpec((1, QT, 32), lambda i, j: (i, j, 0)),
            pl.BlockSpec((1, 32, n), lambda i, j: (i, 0, 0)),
        ],
        out_specs=pl.BlockSpec((1, QT, 1), lambda i, j: (i, j, 0)),
        out_shape=jax.ShapeDtypeStruct((b, q_count, 1), jnp.int32),
    )(qa, w_c)

    idx_flat = idx.reshape(b * q_count)
    q0 = q[:, :, 0].reshape(-1)
    q1 = q[:, :, 1].reshape(-1)
    q2 = q[:, :, 2].reshape(-1)
    s0 = shape[:, :, 0].reshape(-1)
    s1 = shape[:, :, 1].reshape(-1)
    s2 = shape[:, :, 2].reshape(-1)
    n0 = shape_normals[:, :, 0].reshape(-1)
    n1 = shape_normals[:, :, 1].reshape(-1)
    n2 = shape_normals[:, :, 2].reshape(-1)

    sc_loss = _make_sc_loss(b * n, b * q_count)
    partials = sc_loss(idx_flat, q0, q1, q2, s0, s1, s2, n0, n1, n2)

    return jnp.sum(partials) / (b * q_count)


# speedup vs baseline: 4.3873x; 1.0696x over previous
import functools

import jax
import jax.numpy as jnp
from jax import lax
from jax.experimental import pallas as pl
from jax.experimental.pallas import tpu as pltpu
from jax.experimental.pallas import tpu_sc as plsc

INTERP = 10
EPSILON = 0.01

QT = 512
NT = 8192

NW = 32
LANES = 16


def _argmin_kernel(q_ref, w_ref, out_ref):
    i = pl.program_id(0)
    n_total = w_ref.shape[2]

    qa = q_ref[0]

    colf = lax.broadcasted_iota(jnp.int32, (QT, NT), 1).astype(jnp.float32)

    def body(k, carry):
        run_min, run_loc, run_chunk = carry
        sl = pl.ds(k * NT, NT)
        d2m = jnp.dot(qa, w_ref[0, :, sl],
                      preferred_element_type=jnp.float32)
        mn = jnp.min(d2m, axis=1, keepdims=True)
        loc = jnp.min(jnp.where(d2m == mn, colf, float(NT)),
                      axis=1, keepdims=True)

        upd = mn < run_min
        run_loc = jnp.where(upd, loc, run_loc)
        run_chunk = jnp.where(upd, k.astype(jnp.float32), run_chunk)
        run_min = jnp.where(upd, mn, run_min)
        return run_min, run_loc, run_chunk

    init = (jnp.full((QT, 1), jnp.inf, jnp.float32),
            jnp.zeros((QT, 1), jnp.float32),
            jnp.zeros((QT, 1), jnp.float32))
    _, run_loc, run_chunk = lax.fori_loop(0, n_total // NT, body, init)

    idx = (run_chunk * float(NT) + run_loc).astype(jnp.int32) + i * n_total
    out_ref[...] = idx[None]


CHUNK = 64


def _make_sc_loss(total_rows, total_q):
    pw = total_q // NW
    groups = pw // LANES
    chunks = pw // CHUNK
    per_chunk = CHUNK // LANES
    mesh = plsc.VectorSubcoreMesh(core_axis_name="c", subcore_axis_name="s")

    @functools.partial(
        pl.kernel, mesh=mesh,
        out_type=jax.ShapeDtypeStruct((NW, LANES), jnp.float32),
        scratch_types=[
            pltpu.VMEM((chunks, CHUNK), jnp.int32),
            pltpu.VMEM((pw,), jnp.float32),
            pltpu.VMEM((pw,), jnp.float32),
            pltpu.VMEM((pw,), jnp.float32),
            pltpu.VMEM((chunks, CHUNK), jnp.float32),
            pltpu.VMEM((chunks, CHUNK), jnp.float32),
            pltpu.VMEM((chunks, CHUNK), jnp.float32),
            pltpu.VMEM((chunks, CHUNK), jnp.float32),
            pltpu.VMEM((chunks, CHUNK), jnp.float32),
            pltpu.VMEM((chunks, CHUNK), jnp.float32),
            pltpu.VMEM((LANES,), jnp.float32),
            pltpu.SemaphoreType.DMA,
        ],
    )
    def sc_loss(idx_hbm, q0_hbm, q1_hbm, q2_hbm,
                s0_hbm, s1_hbm, s2_hbm, n0_hbm, n1_hbm, n2_hbm,
                out_hbm,
                idx_v, q0_v, q1_v, q2_v,
                s0_g, s1_g, s2_g, n0_g, n1_g, n2_g,
                acc_v, sem):
        wid = lax.axis_index("s") * 2 + lax.axis_index("c")
        base = wid * pw

        for c in range(chunks):
            pltpu.sync_copy(idx_hbm.at[pl.ds(base + c * CHUNK, CHUNK)],
                            idx_v.at[c])
        pltpu.sync_copy(q0_hbm.at[pl.ds(base, pw)], q0_v)
        pltpu.sync_copy(q1_hbm.at[pl.ds(base, pw)], q1_v)
        pltpu.sync_copy(q2_hbm.at[pl.ds(base, pw)], q2_v)

        copies = []
        for c in range(chunks):
            for hbm, dst in ((s0_hbm, s0_g), (s1_hbm, s1_g), (s2_hbm, s2_g),
                             (n0_hbm, n0_g), (n1_hbm, n1_g), (n2_hbm, n2_g)):
                copies.append(
                    pltpu.async_copy(hbm.at[idx_v.at[c]], dst.at[c], sem))
        for cp in copies:
            cp.wait()

        acc = jnp.zeros((LANES,), jnp.float32)
        for g in range(groups):
            r = g // per_chunk
            sl2 = pl.ds((g % per_chunk) * LANES, LANES)
            sl = pl.ds(g * LANES, LANES)
            p0 = s0_g[r, sl2]
            p1 = s1_g[r, sl2]
            p2 = s2_g[r, sl2]
            m0 = n0_g[r, sl2]
            m1 = n1_g[r, sl2]
            m2 = n2_g[r, sl2]
            dot = (((q0_v[sl] - p0) - EPSILON * m0) * m0
                   + ((q1_v[sl] - p1) - EPSILON * m1) * m1
                   + ((q2_v[sl] - p2) - EPSILON * m2) * m2)
            acc = acc + jnp.maximum(-dot, 0.0)

        acc_v[...] = acc
        pltpu.sync_copy(acc_v, out_hbm.at[wid])

    return sc_loss


def kernel(cage, shape, shape_normals):
    b, m, d = cage.shape
    n = shape.shape[1]
    q_count = m * INTERP

    cage_p = jnp.concatenate([cage[:, 1:, :], cage[:, :1, :]], axis=1)
    t = jnp.linspace(0.0, 1.0, INTERP).reshape(1, 1, INTERP, 1)
    q = (t * cage_p[:, :, None, :]
         + (1.0 - t) * cage[:, :, None, :]).reshape(b, q_count, d)

    def _split(x):
        bits = lax.bitcast_convert_type(x, jnp.uint32)
        hi_f = lax.bitcast_convert_type(
            bits & jnp.uint32(0xFFFF0000), jnp.float32)
        return hi_f.astype(jnp.bfloat16), (x - hi_f).astype(jnp.bfloat16)

    shape_t = shape.transpose(0, 2, 1)
    ss = jnp.sum(shape_t * shape_t, axis=1, keepdims=True)
    wf = jnp.concatenate([-2.0 * shape_t, ss,
                          jnp.zeros((b, 4, n), jnp.float32)], axis=1)
    w_hi, w_lo = _split(wf)
    w_c = jnp.concatenate(
        [w_hi, w_lo, w_hi,
         jnp.zeros((b, 8, n), jnp.bfloat16)], axis=1)

    qf = jnp.concatenate(
        [q, jnp.ones((b, q_count, 1), jnp.float32),
         jnp.zeros((b, q_count, 4), jnp.float32)], axis=-1)
    q_hi, q_lo = _split(qf)
    qa = jnp.concatenate(
        [q_hi, q_hi, q_lo,
         jnp.zeros((b, q_count, 8), jnp.bfloat16)], axis=-1)

    idx = pl.pallas_call(
        _argmin_kernel,
        grid=(b, q_count // QT),
        in_specs=[
            pl.BlockSpec((1, QT, 32), lambda i, j: (i, j, 0)),
            pl.BlockSpec((1, 32, n), lambda i, j: (i, 0, 0)),
        ],
        out_specs=pl.BlockSpec((1, QT, 1), lambda i, j: (i, j, 0)),
        out_shape=jax.ShapeDtypeStruct((b, q_count, 1), jnp.int32),
    )(qa, w_c)

    idx_flat = idx.reshape(b * q_count)
    q0 = q[:, :, 0].reshape(-1)
    q1 = q[:, :, 1].reshape(-1)
    q2 = q[:, :, 2].reshape(-1)
    s0 = shape[:, :, 0].reshape(-1)
    s1 = shape[:, :, 1].reshape(-1)
    s2 = shape[:, :, 2].reshape(-1)
    n0 = shape_normals[:, :, 0].reshape(-1)
    n1 = shape_normals[:, :, 1].reshape(-1)
    n2 = shape_normals[:, :, 2].reshape(-1)

    sc_loss = _make_sc_loss(b * n, b * q_count)
    partials = sc_loss(idx_flat, q0, q1, q2, s0, s1, s2, n0, n1, n2)

    return jnp.sum(partials) / (b * q_count)


# speedup vs baseline: 4.4890x; 1.0232x over previous
import functools

import jax
import jax.numpy as jnp
from jax import lax
from jax.experimental import pallas as pl
from jax.experimental.pallas import tpu as pltpu
from jax.experimental.pallas import tpu_sc as plsc

INTERP = 10
EPSILON = 0.01

QT = 1024
NT = 8192

NW = 32
LANES = 16


def _argmin_kernel(q_ref, w_ref, out_ref):
    i = pl.program_id(0)
    n_total = w_ref.shape[2]

    qa = q_ref[0]

    colf = lax.broadcasted_iota(jnp.int32, (QT, NT), 1).astype(jnp.float32)

    def body(k, carry):
        run_min, run_loc, run_chunk = carry
        sl = pl.ds(k * NT, NT)
        d2m = jnp.dot(qa, w_ref[0, :, sl],
                      preferred_element_type=jnp.float32)
        mn = jnp.min(d2m, axis=1, keepdims=True)
        loc = jnp.min(jnp.where(d2m == mn, colf, float(NT)),
                      axis=1, keepdims=True)

        upd = mn < run_min
        run_loc = jnp.where(upd, loc, run_loc)
        run_chunk = jnp.where(upd, k.astype(jnp.float32), run_chunk)
        run_min = jnp.where(upd, mn, run_min)
        return run_min, run_loc, run_chunk

    init = (jnp.full((QT, 1), jnp.inf, jnp.float32),
            jnp.zeros((QT, 1), jnp.float32),
            jnp.zeros((QT, 1), jnp.float32))
    _, run_loc, run_chunk = lax.fori_loop(0, n_total // NT, body, init)

    idx = (run_chunk * float(NT) + run_loc).astype(jnp.int32) + i * n_total
    out_ref[...] = idx[None]


CHUNK = 64


def _make_sc_loss(total_rows, total_q):
    pw = total_q // NW
    groups = pw // LANES
    chunks = pw // CHUNK
    per_chunk = CHUNK // LANES
    mesh = plsc.VectorSubcoreMesh(core_axis_name="c", subcore_axis_name="s")

    @functools.partial(
        pl.kernel, mesh=mesh,
        out_type=jax.ShapeDtypeStruct((NW, LANES), jnp.float32),
        scratch_types=[
            pltpu.VMEM((chunks, CHUNK), jnp.int32),
            pltpu.VMEM((pw,), jnp.float32),
            pltpu.VMEM((pw,), jnp.float32),
            pltpu.VMEM((pw,), jnp.float32),
            pltpu.VMEM((chunks, CHUNK), jnp.float32),
            pltpu.VMEM((chunks, CHUNK), jnp.float32),
            pltpu.VMEM((chunks, CHUNK), jnp.float32),
            pltpu.VMEM((chunks, CHUNK), jnp.float32),
            pltpu.VMEM((chunks, CHUNK), jnp.float32),
            pltpu.VMEM((chunks, CHUNK), jnp.float32),
            pltpu.VMEM((LANES,), jnp.float32),
            pltpu.SemaphoreType.DMA,
        ],
    )
    def sc_loss(idx_hbm, q0_hbm, q1_hbm, q2_hbm,
                s0_hbm, s1_hbm, s2_hbm, n0_hbm, n1_hbm, n2_hbm,
                out_hbm,
                idx_v, q0_v, q1_v, q2_v,
                s0_g, s1_g, s2_g, n0_g, n1_g, n2_g,
                acc_v, sem):
        wid = lax.axis_index("s") * 2 + lax.axis_index("c")
        base = wid * pw

        for c in range(chunks):
            pltpu.sync_copy(idx_hbm.at[pl.ds(base + c * CHUNK, CHUNK)],
                            idx_v.at[c])
        pltpu.sync_copy(q0_hbm.at[pl.ds(base, pw)], q0_v)
        pltpu.sync_copy(q1_hbm.at[pl.ds(base, pw)], q1_v)
        pltpu.sync_copy(q2_hbm.at[pl.ds(base, pw)], q2_v)

        copies = []
        for c in range(chunks):
            for hbm, dst in ((s0_hbm, s0_g), (s1_hbm, s1_g), (s2_hbm, s2_g),
                             (n0_hbm, n0_g), (n1_hbm, n1_g), (n2_hbm, n2_g)):
                copies.append(
                    pltpu.async_copy(hbm.at[idx_v.at[c]], dst.at[c], sem))
        for cp in copies:
            cp.wait()

        acc = jnp.zeros((LANES,), jnp.float32)
        for g in range(groups):
            r = g // per_chunk
            sl2 = pl.ds((g % per_chunk) * LANES, LANES)
            sl = pl.ds(g * LANES, LANES)
            p0 = s0_g[r, sl2]
            p1 = s1_g[r, sl2]
            p2 = s2_g[r, sl2]
            m0 = n0_g[r, sl2]
            m1 = n1_g[r, sl2]
            m2 = n2_g[r, sl2]
            dot = (((q0_v[sl] - p0) - EPSILON * m0) * m0
                   + ((q1_v[sl] - p1) - EPSILON * m1) * m1
                   + ((q2_v[sl] - p2) - EPSILON * m2) * m2)
            acc = acc + jnp.maximum(-dot, 0.0)

        acc_v[...] = acc
        pltpu.sync_copy(acc_v, out_hbm.at[wid])

    return sc_loss


def kernel(cage, shape, shape_normals):
    b, m, d = cage.shape
    n = shape.shape[1]
    q_count = m * INTERP

    cage_p = jnp.concatenate([cage[:, 1:, :], cage[:, :1, :]], axis=1)
    t = jnp.linspace(0.0, 1.0, INTERP).reshape(1, 1, INTERP, 1)
    q = (t * cage_p[:, :, None, :]
         + (1.0 - t) * cage[:, :, None, :]).reshape(b, q_count, d)

    def _split(x):
        bits = lax.bitcast_convert_type(x, jnp.uint32)
        hi_f = lax.bitcast_convert_type(
            bits & jnp.uint32(0xFFFF0000), jnp.float32)
        return hi_f.astype(jnp.bfloat16), (x - hi_f).astype(jnp.bfloat16)

    shape_t = shape.transpose(0, 2, 1)
    ss = jnp.sum(shape_t * shape_t, axis=1, keepdims=True)
    wf = jnp.concatenate([-2.0 * shape_t, ss,
                          jnp.zeros((b, 4, n), jnp.float32)], axis=1)
    w_hi, w_lo = _split(wf)
    w_c = jnp.concatenate(
        [w_hi, w_lo, w_hi,
         jnp.zeros((b, 8, n), jnp.bfloat16)], axis=1)

    qf = jnp.concatenate(
        [q, jnp.ones((b, q_count, 1), jnp.float32),
         jnp.zeros((b, q_count, 4), jnp.float32)], axis=-1)
    q_hi, q_lo = _split(qf)
    qa = jnp.concatenate(
        [q_hi, q_hi, q_lo,
         jnp.zeros((b, q_count, 8), jnp.bfloat16)], axis=-1)

    idx = pl.pallas_call(
        _argmin_kernel,
        grid=(b, q_count // QT),
        in_specs=[
            pl.BlockSpec((1, QT, 32), lambda i, j: (i, j, 0)),
            pl.BlockSpec((1, 32, n), lambda i, j: (i, 0, 0)),
        ],
        out_specs=pl.BlockSpec((1, QT, 1), lambda i, j: (i, j, 0)),
        out_shape=jax.ShapeDtypeStruct((b, q_count, 1), jnp.int32),
    )(qa, w_c)

    idx_flat = idx.reshape(b * q_count)
    q0 = q[:, :, 0].reshape(-1)
    q1 = q[:, :, 1].reshape(-1)
    q2 = q[:, :, 2].reshape(-1)
    s0 = shape[:, :, 0].reshape(-1)
    s1 = shape[:, :, 1].reshape(-1)
    s2 = shape[:, :, 2].reshape(-1)
    n0 = shape_normals[:, :, 0].reshape(-1)
    n1 = shape_normals[:, :, 1].reshape(-1)
    n2 = shape_normals[:, :, 2].reshape(-1)

    sc_loss = _make_sc_loss(b * n, b * q_count)
    partials = sc_loss(idx_flat, q0, q1, q2, s0, s1, s2, n0, n1, n2)

    return jnp.sum(partials) / (b * q_count)


# speedup vs baseline: 4.4986x; 1.0021x over previous
import functools

import jax
import jax.numpy as jnp
from jax import lax
from jax.experimental import pallas as pl
from jax.experimental.pallas import tpu as pltpu
from jax.experimental.pallas import tpu_sc as plsc

INTERP = 10
EPSILON = 0.01

QT = 1024
NT = 8192

NW = 32
LANES = 16


def _argmin_kernel(q_ref, w_ref, out_ref):
    i = pl.program_id(0)
    n_total = w_ref.shape[2]

    qa = q_ref[0]

    colf = lax.broadcasted_iota(jnp.int32, (QT, NT), 1).astype(jnp.float32)

    def body(k, carry):
        run_min, run_loc, run_chunk = carry
        sl = pl.ds(k * NT, NT)
        d2m = jnp.dot(qa, w_ref[0, :, sl],
                      preferred_element_type=jnp.float32)
        mn = jnp.min(d2m, axis=1, keepdims=True)
        loc = jnp.min(jnp.where(d2m == mn, colf, float(NT)),
                      axis=1, keepdims=True)

        upd = mn < run_min
        run_loc = jnp.where(upd, loc, run_loc)
        run_chunk = jnp.where(upd, k.astype(jnp.float32), run_chunk)
        run_min = jnp.where(upd, mn, run_min)
        return run_min, run_loc, run_chunk

    init = (jnp.full((QT, 1), jnp.inf, jnp.float32),
            jnp.zeros((QT, 1), jnp.float32),
            jnp.zeros((QT, 1), jnp.float32))
    _, run_loc, run_chunk = lax.fori_loop(0, n_total // NT, body, init)

    idx = (run_chunk * float(NT) + run_loc).astype(jnp.int32) + i * n_total
    out_ref[...] = idx[None]


CHUNK = 64


def _make_sc_loss(total_rows, total_q):
    pw = total_q // NW
    groups = pw // LANES
    chunks = pw // CHUNK
    per_chunk = CHUNK // LANES
    mesh = plsc.VectorSubcoreMesh(core_axis_name="c", subcore_axis_name="s")

    @functools.partial(
        pl.kernel, mesh=mesh,
        out_type=jax.ShapeDtypeStruct((NW, LANES), jnp.float32),
        scratch_types=[
            pltpu.VMEM((chunks, CHUNK), jnp.int32),
            pltpu.VMEM((pw,), jnp.float32),
            pltpu.VMEM((pw,), jnp.float32),
            pltpu.VMEM((pw,), jnp.float32),
            pltpu.VMEM((chunks, CHUNK), jnp.float32),
            pltpu.VMEM((chunks, CHUNK), jnp.float32),
            pltpu.VMEM((chunks, CHUNK), jnp.float32),
            pltpu.VMEM((chunks, CHUNK), jnp.float32),
            pltpu.VMEM((chunks, CHUNK), jnp.float32),
            pltpu.VMEM((chunks, CHUNK), jnp.float32),
            pltpu.VMEM((LANES,), jnp.float32),
            pltpu.SemaphoreType.DMA,
        ],
    )
    def sc_loss(idx_hbm, q0_hbm, q1_hbm, q2_hbm,
                s0_hbm, s1_hbm, s2_hbm, n0_hbm, n1_hbm, n2_hbm,
                out_hbm,
                idx_v, q0_v, q1_v, q2_v,
                s0_g, s1_g, s2_g, n0_g, n1_g, n2_g,
                acc_v, sem):
        wid = lax.axis_index("s") * 2 + lax.axis_index("c")
        base = wid * pw

        for c in range(chunks):
            pltpu.sync_copy(idx_hbm.at[pl.ds(base + c * CHUNK, CHUNK)],
                            idx_v.at[c])
        pltpu.sync_copy(q0_hbm.at[pl.ds(base, pw)], q0_v)
        pltpu.sync_copy(q1_hbm.at[pl.ds(base, pw)], q1_v)
        pltpu.sync_copy(q2_hbm.at[pl.ds(base, pw)], q2_v)

        copies = []
        for c in range(chunks):
            for hbm, dst in ((s0_hbm, s0_g), (s1_hbm, s1_g), (s2_hbm, s2_g),
                             (n0_hbm, n0_g), (n1_hbm, n1_g), (n2_hbm, n2_g)):
                copies.append(
                    pltpu.async_copy(hbm.at[idx_v.at[c]], dst.at[c], sem))
        for cp in copies:
            cp.wait()

        acc = jnp.zeros((LANES,), jnp.float32)
        for g in range(groups):
            r = g // per_chunk
            sl2 = pl.ds((g % per_chunk) * LANES, LANES)
            sl = pl.ds(g * LANES, LANES)
            p0 = s0_g[r, sl2]
            p1 = s1_g[r, sl2]
            p2 = s2_g[r, sl2]
            m0 = n0_g[r, sl2]
            m1 = n1_g[r, sl2]
            m2 = n2_g[r, sl2]
            dot = (((q0_v[sl] - p0) - EPSILON * m0) * m0
                   + ((q1_v[sl] - p1) - EPSILON * m1) * m1
                   + ((q2_v[sl] - p2) - EPSILON * m2) * m2)
            acc = acc + jnp.maximum(-dot, 0.0)

        acc_v[...] = acc
        pltpu.sync_copy(acc_v, out_hbm.at[wid])

    return sc_loss


def kernel(cage, shape, shape_normals):
    b, m, d = cage.shape
    n = shape.shape[1]
    q_count = m * INTERP

    cage_p = jnp.concatenate([cage[:, 1:, :], cage[:, :1, :]], axis=1)
    t = jnp.linspace(0.0, 1.0, INTERP).reshape(1, 1, INTERP, 1)
    q = (t * cage_p[:, :, None, :]
         + (1.0 - t) * cage[:, :, None, :]).reshape(b, q_count, d)

    def _split(x):
        bits = lax.bitcast_convert_type(x, jnp.uint32)
        hi_f = lax.bitcast_convert_type(
            bits & jnp.uint32(0xFFFF0000), jnp.float32)
        return hi_f.astype(jnp.bfloat16), (x - hi_f).astype(jnp.bfloat16)

    shape_t = shape.transpose(0, 2, 1)
    ss = jnp.sum(shape_t * shape_t, axis=1, keepdims=True)
    wf = jnp.concatenate([-2.0 * shape_t, ss,
                          jnp.zeros((b, 4, n), jnp.float32)], axis=1)
    w_hi, w_lo = _split(wf)
    w_c = jnp.concatenate(
        [w_hi, w_lo, w_hi,
         jnp.zeros((b, 8, n), jnp.bfloat16)], axis=1)

    qf = jnp.concatenate(
        [q, jnp.ones((b, q_count, 1), jnp.float32),
         jnp.zeros((b, q_count, 4), jnp.float32)], axis=-1)
    q_hi, q_lo = _split(qf)
    qa = jnp.concatenate(
        [q_hi, q_hi, q_lo,
         jnp.zeros((b, q_count, 8), jnp.bfloat16)], axis=-1)

    idx = pl.pallas_call(
        _argmin_kernel,
        grid=(b, q_count // QT),
        in_specs=[
            pl.BlockSpec((1, QT, 32), lambda i, j: (i, j, 0)),
            pl.BlockSpec((1, 32, n), lambda i, j: (i, 0, 0)),
        ],
        out_specs=pl.BlockSpec((1, QT, 1), lambda i, j: (i, j, 0)),
        out_shape=jax.ShapeDtypeStruct((b, q_count, 1), jnp.int32),
    )(qa, w_c)

    idx_flat = idx.reshape(b * q_count)
    q_all = q.transpose(2, 0, 1)
    sn_all = jnp.concatenate([shape, shape_normals],
                             axis=2).transpose(2, 0, 1)
    q0 = q_all[0].reshape(-1)
    q1 = q_all[1].reshape(-1)
    q2 = q_all[2].reshape(-1)
    s0 = sn_all[0].reshape(-1)
    s1 = sn_all[1].reshape(-1)
    s2 = sn_all[2].reshape(-1)
    n0 = sn_all[3].reshape(-1)
    n1 = sn_all[4].reshape(-1)
    n2 = sn_all[5].reshape(-1)

    sc_loss = _make_sc_loss(b * n, b * q_count)
    partials = sc_loss(idx_flat, q0, q1, q2, s0, s1, s2, n0, n1, n2)

    return jnp.sum(partials) / (b * q_count)


# speedup vs baseline: 4.5065x; 1.0017x over previous
import functools

import jax
import jax.numpy as jnp
from jax import lax
from jax.experimental import pallas as pl
from jax.experimental.pallas import tpu as pltpu
from jax.experimental.pallas import tpu_sc as plsc

INTERP = 10
EPSILON = 0.01

QT = 1024
NT = 8192

NW = 32
LANES = 16


def _argmin_kernel(q_ref, w_ref, out_ref):
    i = pl.program_id(0)
    n_total = w_ref.shape[2]

    qa = q_ref[0]

    colf = lax.broadcasted_iota(jnp.int32, (QT, NT), 1).astype(jnp.float32)

    def body(k, carry):
        run_min, run_loc, run_chunk = carry
        sl = pl.ds(k * NT, NT)
        d2m = jnp.dot(qa, w_ref[0, :, sl],
                      preferred_element_type=jnp.float32)
        mn = jnp.min(d2m, axis=1, keepdims=True)
        loc = jnp.min(jnp.where(d2m == mn, colf, float(NT)),
                      axis=1, keepdims=True)

        upd = mn < run_min
        run_loc = jnp.where(upd, loc, run_loc)
        run_chunk = jnp.where(upd, k.astype(jnp.float32), run_chunk)
        run_min = jnp.where(upd, mn, run_min)
        return run_min, run_loc, run_chunk

    init = (jnp.full((QT, 1), jnp.inf, jnp.float32),
            jnp.zeros((QT, 1), jnp.float32),
            jnp.zeros((QT, 1), jnp.float32))
    _, run_loc, run_chunk = lax.fori_loop(0, n_total // NT, body, init)

    idx = (run_chunk * float(NT) + run_loc).astype(jnp.int32) + i * n_total
    out_ref[...] = idx[None]


CHUNK = 64


def _make_sc_loss(total_rows, total_q):
    pw = total_q // NW
    groups = pw // LANES
    chunks = pw // CHUNK
    per_chunk = CHUNK // LANES
    mesh = plsc.VectorSubcoreMesh(core_axis_name="c", subcore_axis_name="s")

    @functools.partial(
        pl.kernel, mesh=mesh,
        out_type=jax.ShapeDtypeStruct((NW, LANES), jnp.float32),
        scratch_types=[
            pltpu.VMEM((chunks, CHUNK), jnp.int32),
            pltpu.VMEM((pw,), jnp.float32),
            pltpu.VMEM((pw,), jnp.float32),
            pltpu.VMEM((pw,), jnp.float32),
            pltpu.VMEM((chunks, CHUNK), jnp.float32),
            pltpu.VMEM((chunks, CHUNK), jnp.float32),
            pltpu.VMEM((chunks, CHUNK), jnp.float32),
            pltpu.VMEM((chunks, CHUNK), jnp.float32),
            pltpu.VMEM((chunks, CHUNK), jnp.float32),
            pltpu.VMEM((chunks, CHUNK), jnp.float32),
            pltpu.VMEM((LANES,), jnp.float32),
            pltpu.SemaphoreType.DMA,
        ],
    )
    def sc_loss(idx_hbm, q0_hbm, q1_hbm, q2_hbm,
                s0_hbm, s1_hbm, s2_hbm, n0_hbm, n1_hbm, n2_hbm,
                out_hbm,
                idx_v, q0_v, q1_v, q2_v,
                s0_g, s1_g, s2_g, n0_g, n1_g, n2_g,
                acc_v, sem):
        wid = lax.axis_index("s") * 2 + lax.axis_index("c")
        base = wid * pw

        for c in range(chunks):
            pltpu.sync_copy(idx_hbm.at[pl.ds(base + c * CHUNK, CHUNK)],
                            idx_v.at[c])
        pltpu.sync_copy(q0_hbm.at[pl.ds(base, pw)], q0_v)
        pltpu.sync_copy(q1_hbm.at[pl.ds(base, pw)], q1_v)
        pltpu.sync_copy(q2_hbm.at[pl.ds(base, pw)], q2_v)

        copies = []
        for c in range(chunks):
            for hbm, dst in ((s0_hbm, s0_g), (s1_hbm, s1_g), (s2_hbm, s2_g),
                             (n0_hbm, n0_g), (n1_hbm, n1_g), (n2_hbm, n2_g)):
                copies.append(
                    pltpu.async_copy(hbm.at[idx_v.at[c]], dst.at[c], sem))
        for cp in copies:
            cp.wait()

        acc = jnp.zeros((LANES,), jnp.float32)
        for g in range(groups):
            r = g // per_chunk
            sl2 = pl.ds((g % per_chunk) * LANES, LANES)
            sl = pl.ds(g * LANES, LANES)
            p0 = s0_g[r, sl2]
            p1 = s1_g[r, sl2]
            p2 = s2_g[r, sl2]
            m0 = n0_g[r, sl2]
            m1 = n1_g[r, sl2]
            m2 = n2_g[r, sl2]
            dot = (((q0_v[sl] - p0) - EPSILON * m0) * m0
                   + ((q1_v[sl] - p1) - EPSILON * m1) * m1
                   + ((q2_v[sl] - p2) - EPSILON * m2) * m2)
            acc = acc + jnp.maximum(-dot, 0.0)

        acc_v[...] = acc
        pltpu.sync_copy(acc_v, out_hbm.at[wid])

    return sc_loss


def kernel(cage, shape, shape_normals):
    b, m, d = cage.shape
    n = shape.shape[1]
    q_count = m * INTERP

    cage_p = jnp.concatenate([cage[:, 1:, :], cage[:, :1, :]], axis=1)
    t = jnp.linspace(0.0, 1.0, INTERP).reshape(1, 1, INTERP, 1)
    q = (t * cage_p[:, :, None, :]
         + (1.0 - t) * cage[:, :, None, :]).reshape(b, q_count, d)

    def _split(x):
        bits = lax.bitcast_convert_type(x, jnp.uint32)
        hi_f = lax.bitcast_convert_type(
            bits & jnp.uint32(0xFFFF0000), jnp.float32)
        return hi_f.astype(jnp.bfloat16), (x - hi_f).astype(jnp.bfloat16)

    shape_t = shape.transpose(0, 2, 1)
    ss = jnp.sum(shape_t * shape_t, axis=1, keepdims=True)
    wf = jnp.concatenate([-2.0 * shape_t, ss,
                          jnp.zeros((b, 4, n), jnp.float32)], axis=1)
    w_hi, w_lo = _split(wf)
    w_c = jnp.concatenate(
        [w_hi, w_lo, w_hi, w_lo], axis=1)

    qf = jnp.concatenate(
        [q, jnp.ones((b, q_count, 1), jnp.float32),
         jnp.zeros((b, q_count, 4), jnp.float32)], axis=-1)
    q_hi, q_lo = _split(qf)
    qa = jnp.concatenate(
        [q_hi, q_hi, q_lo, q_lo], axis=-1)

    idx = pl.pallas_call(
        _argmin_kernel,
        grid=(b, q_count // QT),
        in_specs=[
            pl.BlockSpec((1, QT, 32), lambda i, j: (i, j, 0)),
            pl.BlockSpec((1, 32, n), lambda i, j: (i, 0, 0)),
        ],
        out_specs=pl.BlockSpec((1, QT, 1), lambda i, j: (i, j, 0)),
        out_shape=jax.ShapeDtypeStruct((b, q_count, 1), jnp.int32),
    )(qa, w_c)

    idx_flat = idx.reshape(b * q_count)
    q_all = q.transpose(2, 0, 1)
    sn_all = jnp.concatenate([shape, shape_normals],
                             axis=2).transpose(2, 0, 1)
    q0 = q_all[0].reshape(-1)
    q1 = q_all[1].reshape(-1)
    q2 = q_all[2].reshape(-1)
    s0 = sn_all[0].reshape(-1)
    s1 = sn_all[1].reshape(-1)
    s2 = sn_all[2].reshape(-1)
    n0 = sn_all[3].reshape(-1)
    n1 = sn_all[4].reshape(-1)
    n2 = sn_all[5].reshape(-1)

    sc_loss = _make_sc_loss(b * n, b * q_count)
    partials = sc_loss(idx_flat, q0, q1, q2, s0, s1, s2, n0, n1, n2)

    return jnp.sum(partials) / (b * q_count)
